# baseline (speedup 1.0000x reference)
import jax
import jax.numpy as jnp
from jax.experimental import pallas as pl
from jax.experimental.pallas import tpu as pltpu

N = 4096
D = 256
BM = 512
NB = N // BM


def _fused_gcn_kernel(adj_ref, x_ref, w1_ref, b1_ref, w2_ref, b2_ref,
                      o_ref, adjbf_ref, s_ref, h_ref):
    i = pl.program_id(0)

    @pl.when(i == 0)
    def _():
        s_ref[...] = jnp.dot(
            x_ref[...], w1_ref[...], preferred_element_type=jnp.float32
        ).astype(jnp.bfloat16)

    @pl.when(jnp.logical_and(i >= 1, i <= NB))
    def _():
        arow = adjbf_ref[pl.ds((i - 1) * BM, BM), :]
        t = jnp.dot(arow, s_ref[...], preferred_element_type=jnp.float32)
        h_ref[pl.ds((i - 1) * BM, BM), :] = jnp.maximum(
            t + b1_ref[...], 0.0
        ).astype(jnp.bfloat16)

    @pl.when(i < NB)
    def _():
        adjbf_ref[pl.ds(i * BM, BM), :] = adj_ref[...].astype(jnp.bfloat16)

    @pl.when(i == NB + 1)
    def _():
        s_ref[...] = jnp.dot(
            h_ref[...], w2_ref[...], preferred_element_type=jnp.float32
        ).astype(jnp.bfloat16)

    @pl.when(i >= NB + 1)
    def _():
        ab = adjbf_ref[pl.ds((i - NB - 1) * BM, BM), :]
        o_ref[...] = (
            jnp.dot(ab, s_ref[...], preferred_element_type=jnp.float32)
            + b2_ref[...]
        )


def kernel(x, adj, W1, b1, W2, b2):
    xb = x.astype(jnp.bfloat16)
    w1b = W1.astype(jnp.bfloat16)
    w2b = W2.astype(jnp.bfloat16)
    b1r = b1.reshape(1, D)
    b2r = b2.reshape(1, D)
    return pl.pallas_call(
        _fused_gcn_kernel,
        grid=(2 * NB + 1,),
        in_specs=[
            pl.BlockSpec((BM, N), lambda i: (jnp.minimum(i, NB - 1), 0)),
            pl.BlockSpec((N, D), lambda i: (0, 0)),
            pl.BlockSpec((D, D), lambda i: (0, 0)),
            pl.BlockSpec((1, D), lambda i: (0, 0)),
            pl.BlockSpec((D, D), lambda i: (0, 0)),
            pl.BlockSpec((1, D), lambda i: (0, 0)),
        ],
        out_specs=pl.BlockSpec(
            (BM, D), lambda i: (jnp.maximum(i - NB - 1, 0), 0)
        ),
        out_shape=jax.ShapeDtypeStruct((N, D), jnp.float32),
        scratch_shapes=[
            pltpu.VMEM((N, N), jnp.bfloat16),
            pltpu.VMEM((N, D), jnp.bfloat16),
            pltpu.VMEM((N, D), jnp.bfloat16),
        ],
    )(adj, xb, w1b, b1r, w2b, b2r)
